# baseline (speedup 1.0000x reference)
import functools
import math

import jax
import jax.numpy as jnp
from jax import lax
from jax.experimental import pallas as pl
from jax.experimental.pallas import tpu as pltpu
from jax.experimental.pallas import tpu_sc as plsc

N = 10000
N2 = 10240
E = 320000
EP = 327680
F = 128
NS = 16
G = 16
NRB = 8
R_MAX = 5.0
EPS = 0.04

_EB = 2048
_NB = 2048
_NW = 32
_EPW = EP // _NW
_B = 128
_NCH = _EPW // _B
_PH = 2
_NCHP = _NCH // _PH
_NCH0 = 112
_NCH1 = 48
_NCHP0 = _NCH0 // _PH
_NCHP1 = _NCH1 // _PH
_CTAB = 2600
_RPS = N2 // 16
_B2 = 128
_EPW2 = (2 * EP) // _NW
_NCH2 = _EPW2 // _B2
_INTERP = False


def _silu(x):
    s = 1.0 / (1.0 + jnp.exp(-x))
    return x * s


def _dsilu(x):
    s = 1.0 / (1.0 + jnp.exp(-x))
    return s * (1.0 + x * (1.0 - s))


def _dot(a, b):
    return jnp.dot(a, b, preferred_element_type=jnp.float32)


def _geom_T(vT, want_grad):
    eb = vT.shape[1]
    x = vT[0:1, :]
    y = vT[1:2, :]
    z = vT[2:3, :]
    r = jnp.sqrt(x * x + y * y + z * z)
    k = (lax.broadcasted_iota(jnp.int32, (NRB, 1), 0) + 1).astype(jnp.float32)
    c = jnp.float32(math.sqrt(2.0 / R_MAX) * math.pi) * k / R_MAX
    t = k * (r / R_MAX)
    pt = jnp.float32(math.pi) * t
    s = jnp.sin(pt) / (pt + 1e-30)
    bes = c * s
    xs = r / R_MAX
    x2 = xs * xs
    x4 = x2 * x2
    x5 = x4 * xs
    env = 1.0 - 21.0 * x5 + 35.0 * x5 * xs - 15.0 * x5 * x2
    cut = jnp.where(r < R_MAX, env, 0.0)
    rbT = bes * cut
    drbT = None
    if want_grad:
        ds = (jnp.cos(pt) - s) / (t + 1e-30)
        dbes = c * ds * (k / R_MAX)
        denv = (-105.0 * x4 + 210.0 * x5 - 105.0 * x4 * x2) / R_MAX
        dcut = jnp.where(r < R_MAX, denv, 0.0)
        drbT = dbes * cut + bes * dcut
    rp = r + 1e-9
    xh = x / rp
    yh = y / rp
    zh = z / rp
    zeros7 = jnp.zeros((7, eb), jnp.float32)
    sphT = jnp.concatenate(
        [jnp.ones_like(xh), xh, yh, zh, xh * yh, yh * zh, 3.0 * zh * zh - 1.0,
         xh * zh, xh * xh - yh * yh, zeros7], axis=0)
    return r, rbT, drbT, sphT, (xh, yh, zh)


def _mlp_fwd(rb, w0, w1, w2, w3):
    u0 = _dot(rb, w0)
    a0 = _silu(u0)
    u1 = _dot(a0, w1)
    a1 = _silu(u1)
    u2 = _dot(a1, w2)
    a2 = _silu(u2)
    rw = _dot(a2, w3)
    return u0, u1, u2, rw


def _tc_geom(vecsT):
    def body(v_ref, rb_ref, sph_ref):
        _, rbT, _, sphT, _ = _geom_T(v_ref[...], False)
        rb_ref[...] = jnp.swapaxes(rbT, 0, 1)
        sph_ref[...] = jnp.swapaxes(sphT, 0, 1)

    return pl.pallas_call(
        body, grid=(EP // _EB,),
        in_specs=[pl.BlockSpec((3, _EB), lambda i: (0, i))],
        out_specs=[pl.BlockSpec((_EB, NRB), lambda i: (i, 0)),
                   pl.BlockSpec((_EB, 16), lambda i: (i, 0))],
        out_shape=[jax.ShapeDtypeStruct((EP, NRB), jnp.float32),
                   jax.ShapeDtypeStruct((EP, 16), jnp.float32)],
        interpret=_INTERP,
    )(vecsT)


def _tc_edge_fwd(rb, sph, m0, ws0T, m1, ws1T):
    def body(rb_ref, sph_ref, a00, a01, a02, a03, w0T, b00, b01, b02, b03, w1T,
             p0_ref, p1_ref):
        rbv = rb_ref[...]
        sphv = sph_ref[...]
        for refs, wsT, out in (((a00, a01, a02, a03), w0T, p0_ref),
                               ((b00, b01, b02, b03), w1T, p1_ref)):
            _, _, _, rw = _mlp_fwd(rbv, refs[0][...], refs[1][...], refs[2][...], refs[3][...])
            se = _dot(sphv, wsT[...])
            out[...] = rw * se

    full = lambda i: (0, 0)
    specs = [pl.BlockSpec((_EB, NRB), lambda i: (i, 0)),
             pl.BlockSpec((_EB, 16), lambda i: (i, 0))]
    for shp in ((NRB, 64), (64, 64), (64, 64), (64, F), (16, 1),
                (NRB, 64), (64, 64), (64, 64), (64, F), (16, 1)):
        specs.append(pl.BlockSpec(shp, full))
    out_spec = pl.BlockSpec((_EB, F), lambda i: (i, 0))
    return pl.pallas_call(
        body, grid=(EP // _EB,), in_specs=specs,
        out_specs=[out_spec, out_spec],
        out_shape=[jax.ShapeDtypeStruct((EP, F), jnp.float32)] * 2,
        interpret=_INTERP,
    )(rb, sph, *m0, ws0T, *m1, ws1T)


def _tc_node_embed(species2d, w_emb, W_up0):
    def body(sp_ref, emb_ref, wu_ref, a0_ref):
        sp = sp_ref[...]
        oh = (sp == lax.broadcasted_iota(jnp.int32, (_NB, NS), 1)).astype(jnp.float32)
        nf0 = _dot(oh, emb_ref[...]) * jnp.float32(1.0 / math.sqrt(NS))
        a0_ref[...] = _dot(nf0, wu_ref[...])

    return pl.pallas_call(
        body, grid=(N2 // _NB,),
        in_specs=[pl.BlockSpec((_NB, 1), lambda i: (i, 0)),
                  pl.BlockSpec((NS, F), lambda i: (0, 0)),
                  pl.BlockSpec((F, F), lambda i: (0, 0))],
        out_specs=pl.BlockSpec((_NB, F), lambda i: (i, 0)),
        out_shape=jax.ShapeDtypeStruct((N2, F), jnp.float32),
        interpret=_INTERP,
    )(species2d, w_emb, W_up0)


def _onehot(sp):
    return (sp == lax.broadcasted_iota(jnp.int32, (sp.shape[0], NS), 1)).astype(jnp.float32)


def _species_mm(oh, x, wz_flat_ref):
    acc = jnp.zeros_like(x)
    for s in range(NS):
        acc = acc + oh[:, s:s + 1] * _dot(x, wz_flat_ref[s * F:(s + 1) * F, :])
    return acc


def _tc_node_l0(p00, p01, sp2d, W_down0, Wz0f, wsc0a, wsc0b, wsc0c, W_post0, W_up1, Wro0):
    def body(pa, pb, sp_ref, wd, wz, wa, wb, wc, wp, wu1, wro,
             nf1_ref, a1_ref, c0_ref, es0_ref):
        oh = _onehot(sp_ref[...])
        agg = (pa[...] + pb[...]) * jnp.float32(EPS)
        B0 = _dot(agg, wd[...])
        C0 = _species_mm(oh, B0, wz)
        wa_n = _dot(oh, wa[...])
        wb_n = _dot(oh, wb[...])
        wc_n = _dot(oh, wc[...])
        D0 = wa_n * C0 + wb_n * C0 * C0 + wc_n * C0 * C0 * C0
        nf1 = _dot(D0, wp[...])
        nf1_ref[...] = nf1
        a1_ref[...] = _dot(nf1, wu1[...])
        c0_ref[...] = C0
        es0_ref[...] = _dot(nf1, wro[...])

    nb = pl.BlockSpec((_NB, F), lambda i: (i, 0))
    full = lambda i: (0, 0)
    return pl.pallas_call(
        body, grid=(N2 // _NB,),
        in_specs=[nb, nb, pl.BlockSpec((_NB, 1), lambda i: (i, 0)),
                  pl.BlockSpec((F, F), full), pl.BlockSpec((NS * F, F), full),
                  pl.BlockSpec((NS, F), full), pl.BlockSpec((NS, F), full),
                  pl.BlockSpec((NS, F), full), pl.BlockSpec((F, F), full),
                  pl.BlockSpec((F, F), full), pl.BlockSpec((F, 1), full)],
        out_specs=[nb, nb, nb, pl.BlockSpec((_NB, 1), lambda i: (i, 0))],
        out_shape=[jax.ShapeDtypeStruct((N2, F), jnp.float32),
                   jax.ShapeDtypeStruct((N2, F), jnp.float32),
                   jax.ShapeDtypeStruct((N2, F), jnp.float32),
                   jax.ShapeDtypeStruct((N2, 1), jnp.float32)],
        interpret=_INTERP,
    )(p00, p01, sp2d, W_down0, Wz0f, wsc0a, wsc0b, wsc0c, W_post0, W_up1, Wro0)


def _tc_node_l1(p10, p11, nf1, sp2d, inde2d, es0, offsets2d, W_down1, Wz1f, Wz1Tf,
                wsc1a, wsc1b, wsc1c, W_post1, W_post1T, W_down1T, Wr1, Wr1T, Wr2, Wr2T):
    def body(pa, pb, nf_ref, sp_ref, ge_ref, es0_ref, off, wd, wz, wzT,
             wa, wb, wc, wp, wpT, wdT, wr1, wr1T, wr2, wr2T,
             gagg_ref, gnf_ref, eg_ref):
        oh = _onehot(sp_ref[...])
        nf = nf_ref[...]
        agg = (pa[...] + pb[...]) * jnp.float32(EPS)
        B1 = _dot(agg, wd[...])
        wa_n = _dot(oh, wa[...])
        wb_n = _dot(oh, wb[...])
        wc_n = _dot(oh, wc[...])
        D1 = wa_n * B1 + wb_n * B1 * B1 + wc_n * B1 * B1 * B1
        H1 = _dot(D1, wp[...]) + _species_mm(oh, nf, wz)
        u = _dot(H1, wr1[...])
        out1 = _dot(_silu(u), wr2[...])
        Ei = es0_ref[...] + out1 + _dot(oh, off[...])
        goh = (ge_ref[...] == lax.broadcasted_iota(jnp.int32, (_NB, G), 1)).astype(jnp.float32)
        eg_ref[...] = jnp.sum(goh * Ei, axis=0, keepdims=True)[None]
        gu = _dsilu(u) * wr2T[...]
        gH1 = _dot(gu, wr1T[...])
        gD1 = _dot(gH1, wpT[...])
        gnf_ref[...] = _species_mm(oh, gH1, wzT)
        gB1 = gD1 * (wa_n + 2.0 * wb_n * B1 + 3.0 * wc_n * B1 * B1)
        gagg_ref[...] = _dot(gB1, wdT[...]) * jnp.float32(EPS)

    nb = pl.BlockSpec((_NB, F), lambda i: (i, 0))
    n1 = pl.BlockSpec((_NB, 1), lambda i: (i, 0))
    full = lambda i: (0, 0)
    f3 = lambda i: (i, 0, 0)
    return pl.pallas_call(
        body, grid=(N2 // _NB,),
        in_specs=[nb, nb, nb, n1, n1, n1, pl.BlockSpec((NS, 1), full),
                  pl.BlockSpec((F, F), full), pl.BlockSpec((NS * F, F), full),
                  pl.BlockSpec((NS * F, F), full),
                  pl.BlockSpec((NS, F), full), pl.BlockSpec((NS, F), full),
                  pl.BlockSpec((NS, F), full), pl.BlockSpec((F, F), full),
                  pl.BlockSpec((F, F), full), pl.BlockSpec((F, F), full),
                  pl.BlockSpec((F, 16), full), pl.BlockSpec((16, F), full),
                  pl.BlockSpec((16, 1), full), pl.BlockSpec((1, 16), full)],
        out_specs=[nb, nb, pl.BlockSpec((1, 1, G), f3)],
        out_shape=[jax.ShapeDtypeStruct((N2, F), jnp.float32),
                   jax.ShapeDtypeStruct((N2, F), jnp.float32),
                   jax.ShapeDtypeStruct((N2 // _NB, 1, G), jnp.float32)],
        interpret=_INTERP,
    )(p10, p11, nf1, sp2d, inde2d, es0, offsets2d, W_down1, Wz1f, Wz1Tf,
      wsc1a, wsc1b, wsc1c, W_post1, W_post1T, W_down1T, Wr1, Wr1T, Wr2, Wr2T)


def _tc_node_l0_bwd(gnf1a, ga10, ga11, c0, sp2d, W_up1T, wro0row, W_post0T,
                    Wz0Tf, wsc0a, wsc0b, wsc0c, W_down0T):
    def body(gn_ref, ga, gb, c0_ref, sp_ref, wu1T, wro, wpT, wzT, wa, wb, wc, wdT,
             gagg_ref):
        oh = _onehot(sp_ref[...])
        C0 = c0_ref[...]
        gnf1 = gn_ref[...] + _dot(ga[...] + gb[...], wu1T[...]) + wro[...]
        gD0 = _dot(gnf1, wpT[...])
        wa_n = _dot(oh, wa[...])
        wb_n = _dot(oh, wb[...])
        wc_n = _dot(oh, wc[...])
        gC0 = gD0 * (wa_n + 2.0 * wb_n * C0 + 3.0 * wc_n * C0 * C0)
        gB0 = _species_mm(oh, gC0, wzT)
        gagg_ref[...] = _dot(gB0, wdT[...]) * jnp.float32(EPS)

    nb = pl.BlockSpec((_NB, F), lambda i: (i, 0))
    full = lambda i: (0, 0)
    return pl.pallas_call(
        body, grid=(N2 // _NB,),
        in_specs=[nb, nb, nb, nb, pl.BlockSpec((_NB, 1), lambda i: (i, 0)),
                  pl.BlockSpec((F, F), full), pl.BlockSpec((1, F), full),
                  pl.BlockSpec((F, F), full), pl.BlockSpec((NS * F, F), full),
                  pl.BlockSpec((NS, F), full), pl.BlockSpec((NS, F), full),
                  pl.BlockSpec((NS, F), full), pl.BlockSpec((F, F), full)],
        out_specs=nb,
        out_shape=jax.ShapeDtypeStruct((N2, F), jnp.float32),
        interpret=_INTERP,
    )(gnf1a, ga10, ga11, c0, sp2d, W_up1T, wro0row, W_post0T, Wz0Tf,
      wsc0a, wsc0b, wsc0c, W_down0T)


def _tc_edge_bwd_force(vecsT, rb, sph, gp0, gp1, maskT, m0, m0T, ws0T, m1, m1T, ws1T):
    def body(v_ref, rb_ref, sph_ref, g0_ref, g1_ref, mk_ref,
             a00, a01, a02, a03, ta0, ta1, ta2, ta3, w0T,
             b00, b01, b02, b03, tb0, tb1, tb2, tb3, w1T,
             gvp_ref, gvn_ref):
        rbv = rb_ref[...]
        sphv = sph_ref[...]
        grb = jnp.zeros((_EB, NRB), jnp.float32)
        gses = []
        for fw, bw, wsT, g_ref in (
                ((a00, a01, a02, a03), (ta0, ta1, ta2, ta3), w0T, g0_ref),
                ((b00, b01, b02, b03), (tb0, tb1, tb2, tb3), w1T, g1_ref)):
            u0, u1, u2, rw = _mlp_fwd(rbv, fw[0][...], fw[1][...], fw[2][...], fw[3][...])
            se = _dot(sphv, wsT[...])
            gp = g_ref[...]
            grw = gp * se
            gses.append(jnp.sum(gp * rw, axis=1, keepdims=True))
            ga2 = _dot(grw, bw[3][...])
            gu2 = ga2 * _dsilu(u2)
            ga1 = _dot(gu2, bw[2][...])
            gu1 = ga1 * _dsilu(u1)
            ga0 = _dot(gu1, bw[1][...])
            gu0 = ga0 * _dsilu(u0)
            grb = grb + _dot(gu0, bw[0][...])
        vT = v_ref[...]
        r, _, drbT, _, vh = _geom_T(vT, True)
        grbT = jnp.swapaxes(grb, 0, 1)
        gse0T = jnp.swapaxes(gses[0], 0, 1)
        gse1T = jnp.swapaxes(gses[1], 0, 1)
        gr = jnp.sum(grbT * drbT, axis=0, keepdims=True)
        gsphT = w0T[...] * gse0T + w1T[...] * gse1T
        xh, yh, zh = vh
        gx = gsphT[1:2] + gsphT[4:5] * yh + gsphT[7:8] * zh + 2.0 * xh * gsphT[8:9]
        gy = gsphT[2:3] + gsphT[4:5] * xh + gsphT[5:6] * zh - 2.0 * yh * gsphT[8:9]
        gz = gsphT[3:4] + gsphT[5:6] * yh + 6.0 * zh * gsphT[6:7] + gsphT[7:8] * xh
        rp = r + 1e-9
        vx = vT[0:1, :]
        vy = vT[1:2, :]
        vz = vT[2:3, :]
        vdg = (vx * gx + vy * gy + vz * gz) / (rp * rp * r)
        s1 = gr / r
        mk = mk_ref[...]
        gvx = (s1 * vx + gx / rp - vx * vdg) * mk
        gvy = (s1 * vy + gy / rp - vy * vdg) * mk
        gvz = (s1 * vz + gz / rp - vz * vdg) * mk
        z13 = jnp.zeros((13, _EB), jnp.float32)
        gvT = jnp.concatenate([gvx, gvy, gvz, z13], axis=0)
        gvp_ref[...] = jnp.swapaxes(gvT, 0, 1)
        gvn_ref[...] = jnp.swapaxes(-gvT, 0, 1)

    eb = pl.BlockSpec((_EB, F), lambda i: (i, 0))
    o16 = pl.BlockSpec((_EB, 16), lambda i: (i, 0))
    full = lambda i: (0, 0)
    specs = [pl.BlockSpec((3, _EB), lambda i: (0, i)),
             pl.BlockSpec((_EB, NRB), lambda i: (i, 0)), o16, eb, eb,
             pl.BlockSpec((1, _EB), lambda i: (0, i))]
    for shp in ((NRB, 64), (64, 64), (64, 64), (64, F),
                (64, NRB), (64, 64), (64, 64), (F, 64), (16, 1)) * 2:
        specs.append(pl.BlockSpec(shp, full))
    return pl.pallas_call(
        body, grid=(EP // _EB,), in_specs=specs,
        out_specs=[o16, o16],
        out_shape=[jax.ShapeDtypeStruct((EP, 16), jnp.float32)] * 2,
        interpret=_INTERP,
    )(vecsT, rb, sph, gp0, gp1, maskT, *m0, *m0T, ws0T, *m1, *m1T, ws1T)


_MESH = dict(core_axis_name="c", subcore_axis_name="s")


def _wid():
    return lax.axis_index("s") * 2 + lax.axis_index("c")


def _split_sched(nch0=_NCH0):
    cid = lax.axis_index("c")
    sid = lax.axis_index("s")
    nchp = jnp.where(cid == 0, nch0 // _PH, (_NCH0 + _NCH1 - nch0) // _PH)
    base = sid * (_NCH0 + _NCH1) + cid * nch0

    def tab_base(ph):
        return base + ph * nchp

    return nchp, tab_base


def _mul_rows(dst, src, nrows):
    @plsc.parallel_loop(0, nrows, unroll=2)
    def mul(e):
        for j in range(F // 16):
            sl = pl.ds(j * 16, 16)
            dst[e, sl] = dst[e, sl] * src[e, sl]


def _sc_fwd(A, P, ia2, ib2, zeros_nf):
    mesh = plsc.VectorSubcoreMesh(**_MESH)

    @functools.partial(
        pl.kernel, mesh=mesh,
        out_type=jax.ShapeDtypeStruct((2, N2, F), jnp.float32),
        scratch_types=[
            pltpu.VMEM((_NCHP0, _B), jnp.int32),
            pltpu.VMEM((_NCHP0, _B), jnp.int32),
            pltpu.VMEM((_B, F), jnp.float32),
            pltpu.VMEM((_B, F), jnp.float32),
            pltpu.VMEM_SHARED((N2, F), jnp.float32),
            pltpu.SemaphoreType.DMA,
            pltpu.SemaphoreType.DMA,
        ],
    )
    def k(a_hbm, p_hbm, ia_hbm, ib_hbm, z_hbm, out_hbm, ia_v, ib_v, rows_v, p_v,
          acc, sem, semp):
        cid = lax.axis_index("c")
        sid = lax.axis_index("s")
        rbase = sid * _RPS
        pltpu.sync_copy(z_hbm.at[pl.ds(rbase, _RPS)], acc.at[pl.ds(rbase, _RPS)])
        plsc.subcore_barrier()
        nchp, tab_base = _split_sched()

        for ph in range(_PH):
            tb = tab_base(ph)
            pltpu.sync_copy(ia_hbm.at[pl.ds(tb, _NCHP0)], ia_v)
            pltpu.sync_copy(ib_hbm.at[pl.ds(tb, _NCHP0)], ib_v)

            def chunk(ci, _):
                b = (tb + ci) * _B
                cp = pltpu.async_copy(p_hbm.at[pl.ds(b, _B)], p_v, semp)
                pltpu.async_copy(a_hbm.at[ia_v.at[ci]], rows_v, sem).wait()
                cp.wait()
                _mul_rows(p_v, rows_v, _B)
                pltpu.sync_copy(p_v, acc.at[ib_v.at[ci]], add=True)
                return 0

            lax.fori_loop(0, nchp, chunk, 0)
        plsc.subcore_barrier()
        pltpu.sync_copy(acc.at[pl.ds(rbase, _RPS)],
                        out_hbm.at[cid, pl.ds(rbase, _RPS)])

    return k(A, P, ia2, ib2, zeros_nf)


def _sc_bwd1(Gs, A, P, ia2, ib2, zeros_nf):
    mesh = plsc.VectorSubcoreMesh(**_MESH)

    @functools.partial(
        pl.kernel, mesh=mesh,
        out_type=[jax.ShapeDtypeStruct((EP, F), jnp.float32),
                  jax.ShapeDtypeStruct((2, N2, F), jnp.float32)],
        scratch_types=[
            pltpu.VMEM((_NCHP0, _B), jnp.int32),
            pltpu.VMEM((_NCHP0, _B), jnp.int32),
            pltpu.VMEM((_B, F), jnp.float32),
            pltpu.VMEM((_B, F), jnp.float32),
            pltpu.VMEM_SHARED((N2, F), jnp.float32),
            pltpu.SemaphoreType.DMA,
            pltpu.SemaphoreType.DMA,
            pltpu.SemaphoreType.DMA,
        ],
    )
    def k(g_hbm, a_hbm, p_hbm, ia_hbm, ib_hbm, z_hbm, gp_hbm, out_hbm,
          ia_v, ib_v, q_v, a_v, acc, sem, semq, semp):
        cid = lax.axis_index("c")
        sid = lax.axis_index("s")
        rbase = sid * _RPS
        pltpu.sync_copy(z_hbm.at[pl.ds(rbase, _RPS)], acc.at[pl.ds(rbase, _RPS)])
        plsc.subcore_barrier()
        nchp, tab_base = _split_sched()

        for ph in range(_PH):
            tb = tab_base(ph)
            pltpu.sync_copy(ia_hbm.at[pl.ds(tb, _NCHP0)], ia_v)
            pltpu.sync_copy(ib_hbm.at[pl.ds(tb, _NCHP0)], ib_v)

            def chunk(ci, _):
                b = (tb + ci) * _B
                cq = pltpu.async_copy(g_hbm.at[ib_v.at[ci]], q_v, semq)
                pltpu.async_copy(a_hbm.at[ia_v.at[ci]], a_v, sem).wait()
                cq.wait()
                _mul_rows(a_v, q_v, _B)
                pltpu.sync_copy(a_v, gp_hbm.at[pl.ds(b, _B)])
                pltpu.async_copy(p_hbm.at[pl.ds(b, _B)], a_v, semp).wait()
                _mul_rows(a_v, q_v, _B)
                pltpu.sync_copy(a_v, acc.at[ia_v.at[ci]], add=True)
                return 0

            lax.fori_loop(0, nchp, chunk, 0)
        plsc.subcore_barrier()
        pltpu.sync_copy(acc.at[pl.ds(rbase, _RPS)],
                        out_hbm.at[cid, pl.ds(rbase, _RPS)])

    return k(Gs, A, P, ia2, ib2, zeros_nf)


def _sc_bwd0(Gs, A, ia2, ib2):
    mesh = plsc.VectorSubcoreMesh(**_MESH)

    @functools.partial(
        pl.kernel, mesh=mesh,
        out_type=jax.ShapeDtypeStruct((EP, F), jnp.float32),
        scratch_types=[
            pltpu.VMEM((_NCHP0, _B), jnp.int32),
            pltpu.VMEM((_NCHP0, _B), jnp.int32),
            pltpu.VMEM((_B, F), jnp.float32),
            pltpu.VMEM((_B, F), jnp.float32),
            pltpu.SemaphoreType.DMA,
            pltpu.SemaphoreType.DMA,
        ],
    )
    def k(g_hbm, a_hbm, ia_hbm, ib_hbm, gp_hbm, ia_v, ib_v, q_v, a_v, sem, semq):
        nchp, tab_base = _split_sched()

        for ph in range(_PH):
            tb = tab_base(ph)
            pltpu.sync_copy(ia_hbm.at[pl.ds(tb, _NCHP0)], ia_v)
            pltpu.sync_copy(ib_hbm.at[pl.ds(tb, _NCHP0)], ib_v)

            def chunk(ci, _):
                b = (tb + ci) * _B
                cq = pltpu.async_copy(g_hbm.at[ib_v.at[ci]], q_v, semq)
                pltpu.async_copy(a_hbm.at[ia_v.at[ci]], a_v, sem).wait()
                cq.wait()
                _mul_rows(a_v, q_v, _B)
                pltpu.sync_copy(a_v, gp_hbm.at[pl.ds(b, _B)])
                return 0

            lax.fori_loop(0, nchp, chunk, 0)

    return k(Gs, A, ia2, ib2)


def _sc_forces(idx2, val2, zeros_nf):
    mesh = plsc.VectorSubcoreMesh(**_MESH)

    @functools.partial(
        pl.kernel, mesh=mesh,
        out_type=jax.ShapeDtypeStruct((2, N2, F), jnp.float32),
        scratch_types=[
            pltpu.VMEM((_NCH2 // _PH, _B2), jnp.int32),
            pltpu.VMEM((_B2, 16), jnp.float32),
            pltpu.VMEM((_B2, F), jnp.float32),
            pltpu.VMEM_SHARED((N2, F), jnp.float32),
            pltpu.SemaphoreType.DMA,
        ],
    )
    def k(ix_hbm, v_hbm, z_hbm, out_hbm, ix_v, v_v, w_v, acc, semv):
        cid = lax.axis_index("c")
        sid = lax.axis_index("s")
        rbase = sid * _RPS
        pltpu.sync_copy(z_hbm.at[pl.ds(rbase, _RPS)], acc.at[pl.ds(rbase, _RPS)])
        w = _wid()
        zv = jnp.zeros((16,), jnp.float32)

        @plsc.parallel_loop(0, _B2, unroll=2)
        def zrow(e):
            for j in range(F // 16):
                w_v[e, pl.ds(j * 16, 16)] = zv
        plsc.subcore_barrier()
        nchp2 = _NCH2 // _PH

        for ph in range(_PH):
            pltpu.sync_copy(ix_hbm.at[pl.ds(w * _NCH2 + ph * nchp2, nchp2)], ix_v)
            ebase = w * _EPW2 + ph * nchp2 * _B2

            def chunk(ci, _):
                b = ebase + ci * _B2
                pltpu.async_copy(v_hbm.at[pl.ds(b, _B2)], v_v, semv).wait()

                @plsc.parallel_loop(0, _B2, unroll=4)
                def crow(e):
                    w_v[e, pl.ds(0, 16)] = v_v[e, pl.ds(0, 16)]
                pltpu.sync_copy(w_v, acc.at[ix_v.at[ci]], add=True)
                return 0

            lax.fori_loop(0, nchp2, chunk, 0)
        plsc.subcore_barrier()
        pltpu.sync_copy(acc.at[pl.ds(rbase, _RPS)],
                        out_hbm.at[cid, pl.ds(rbase, _RPS)])

    return k(idx2, val2, zeros_nf)


def kernel(nn_vecs, species, inda, indb, inde, nats, mask, w_emb, offsets,
           l0_W_up, l0_mlp0, l0_mlp1, l0_mlp2, l0_mlp3, l0_w_s, l0_W_down,
           l0_Wz, l0_w_sc, l0_W_post,
           l1_W_up, l1_mlp0, l1_mlp1, l1_mlp2, l1_mlp3, l1_w_s, l1_W_down,
           l1_Wz, l1_w_sc, l1_W_post,
           Wro0, Wr1, Wr2):
    f32 = jnp.float32
    sp2d = jnp.pad(species.astype(jnp.int32), (0, N2 - N)).reshape(N2, 1)
    ia = jnp.pad(inda.astype(jnp.int32), (0, EP - E), constant_values=N)
    ib = jnp.pad(indb.astype(jnp.int32), (0, EP - E), constant_values=N)
    ia2 = jnp.pad(ia.reshape(EP // _B, _B), ((0, _CTAB - EP // _B), (0, 0)))
    ib2 = jnp.pad(ib.reshape(EP // _B, _B), ((0, _CTAB - EP // _B), (0, 0)))
    inde2d = jnp.pad(inde.astype(jnp.int32), (0, N2 - N), constant_values=G).reshape(N2, 1)
    maskT = jnp.pad(mask.astype(f32), (0, EP - E)).reshape(1, EP)
    vecsT = jnp.pad(nn_vecs, ((0, EP - E), (0, 0)), constant_values=1.0).T
    zeros_nf = jnp.zeros((N2, F), f32)

    def pad16(w):
        return jnp.concatenate([w, jnp.zeros((7,), f32)]).reshape(16, 1)

    ws0T, ws1T = pad16(l0_w_s), pad16(l1_w_s)
    m0 = (l0_mlp0, l0_mlp1, l0_mlp2, l0_mlp3)
    m1 = (l1_mlp0, l1_mlp1, l1_mlp2, l1_mlp3)
    m0T = tuple(w.T for w in m0)
    m1T = tuple(w.T for w in m1)
    Wz0f = l0_Wz.reshape(NS * F, F)
    Wz1f = l1_Wz.reshape(NS * F, F)
    Wz0Tf = jnp.swapaxes(l0_Wz, 1, 2).reshape(NS * F, F)
    Wz1Tf = jnp.swapaxes(l1_Wz, 1, 2).reshape(NS * F, F)
    wsc0a, wsc0b, wsc0c = l0_w_sc[:, 0], l0_w_sc[:, 1], l0_w_sc[:, 2]
    wsc1a, wsc1b, wsc1c = l1_w_sc[:, 0], l1_w_sc[:, 1], l1_w_sc[:, 2]

    rb, sph = _tc_geom(vecsT)
    P0, P1 = _tc_edge_fwd(rb, sph, m0, ws0T, m1, ws1T)
    A0 = _tc_node_embed(sp2d, w_emb, l0_W_up)
    agg0p = _sc_fwd(A0, P0, ia2, ib2, zeros_nf)
    nf1, A1, C0, Es0 = _tc_node_l0(agg0p[0], agg0p[1], sp2d, l0_W_down, Wz0f,
                                   wsc0a, wsc0b, wsc0c, l0_W_post, l1_W_up, Wro0)
    agg1p = _sc_fwd(A1, P1, ia2, ib2, zeros_nf)
    gagg1, gnf1a, egp = _tc_node_l1(
        agg1p[0], agg1p[1], nf1, sp2d, inde2d, Es0, offsets.reshape(NS, 1),
        l1_W_down, Wz1f, Wz1Tf, wsc1a, wsc1b, wsc1c, l1_W_post, l1_W_post.T,
        l1_W_down.T, Wr1, Wr1.T, Wr2, Wr2.T)
    Eg = jnp.sum(egp, axis=(0, 1))

    gP1, ga1p = _sc_bwd1(gagg1, A1, P1, ia2, ib2, zeros_nf)
    gagg0 = _tc_node_l0_bwd(gnf1a, ga1p[0], ga1p[1], C0, sp2d, l1_W_up.T,
                            Wro0.reshape(1, F), l0_W_post.T, Wz0Tf,
                            wsc0a, wsc0b, wsc0c, l0_W_down.T)
    gP0 = _sc_bwd0(gagg0, A0, ia2, ib2)
    gvp, gvn = _tc_edge_bwd_force(vecsT, rb, sph, gP0, gP1, maskT,
                                  m0, m0T, ws0T, m1, m1T, ws1T)
    idx2 = jnp.concatenate([ia, ib]).reshape((2 * EP) // _B2, _B2)
    val2 = jnp.concatenate([gvp, gvn], axis=0)
    fp = _sc_forces(idx2, val2, zeros_nf)
    Fn = (fp[0] + fp[1])[:N, :3]
    return (Eg, Fn)


# speedup vs baseline: 1.0259x; 1.0259x over previous
import functools
import math

import jax
import jax.numpy as jnp
from jax import lax
from jax.experimental import pallas as pl
from jax.experimental.pallas import tpu as pltpu
from jax.experimental.pallas import tpu_sc as plsc

N = 10000
N2 = 10240
E = 320000
EP = 327680
F = 128
NS = 16
G = 16
NRB = 8
R_MAX = 5.0
EPS = 0.04

_EB = 2048
_NB = 2048
_NW = 32
_EPW = EP // _NW
_B = 128
_NCH = _EPW // _B
_PH = 2
_NCHP = _NCH // _PH
_NCH0 = 112
_NCH1 = 48
_NCHP0 = _NCH0 // _PH
_NCHP1 = _NCH1 // _PH
_CTAB = 2600
_RPS = N2 // 16
_B2 = 128
_EPW2 = (2 * EP) // _NW
_NCH2 = _EPW2 // _B2


def _silu(x):
    s = 1.0 / (1.0 + jnp.exp(-x))
    return x * s


def _dsilu(x):
    s = 1.0 / (1.0 + jnp.exp(-x))
    return s * (1.0 + x * (1.0 - s))


def _dot(a, b):
    return jnp.dot(a, b, preferred_element_type=jnp.float32)


def _geom_T(vT, want_grad):
    eb = vT.shape[1]
    x = vT[0:1, :]
    y = vT[1:2, :]
    z = vT[2:3, :]
    r = jnp.sqrt(x * x + y * y + z * z)
    k = (lax.broadcasted_iota(jnp.int32, (NRB, 1), 0) + 1).astype(jnp.float32)
    c = jnp.float32(math.sqrt(2.0 / R_MAX) * math.pi) * k / R_MAX
    t = k * (r / R_MAX)
    pt = jnp.float32(math.pi) * t
    s = jnp.sin(pt) / (pt + 1e-30)
    bes = c * s
    xs = r / R_MAX
    x2 = xs * xs
    x4 = x2 * x2
    x5 = x4 * xs
    env = 1.0 - 21.0 * x5 + 35.0 * x5 * xs - 15.0 * x5 * x2
    cut = jnp.where(r < R_MAX, env, 0.0)
    rbT = bes * cut
    drbT = None
    if want_grad:
        ds = (jnp.cos(pt) - s) / (t + 1e-30)
        dbes = c * ds * (k / R_MAX)
        denv = (-105.0 * x4 + 210.0 * x5 - 105.0 * x4 * x2) / R_MAX
        dcut = jnp.where(r < R_MAX, denv, 0.0)
        drbT = dbes * cut + bes * dcut
    rp = r + 1e-9
    xh = x / rp
    yh = y / rp
    zh = z / rp
    zeros7 = jnp.zeros((7, eb), jnp.float32)
    sphT = jnp.concatenate(
        [jnp.ones_like(xh), xh, yh, zh, xh * yh, yh * zh, 3.0 * zh * zh - 1.0,
         xh * zh, xh * xh - yh * yh, zeros7], axis=0)
    return r, rbT, drbT, sphT, (xh, yh, zh)


def _mlp_fwd(rb, w0, w1, w2, w3):
    u0 = _dot(rb, w0)
    a0 = _silu(u0)
    u1 = _dot(a0, w1)
    a1 = _silu(u1)
    u2 = _dot(a1, w2)
    a2 = _silu(u2)
    rw = _dot(a2, w3)
    return u0, u1, u2, rw


def _tc_geom(vecsT):
    def body(v_ref, rb_ref, sph_ref):
        _, rbT, _, sphT, _ = _geom_T(v_ref[...], False)
        rb_ref[...] = jnp.swapaxes(rbT, 0, 1)
        sph_ref[...] = jnp.swapaxes(sphT, 0, 1)

    return pl.pallas_call(
        body, grid=(EP // _EB,),
        in_specs=[pl.BlockSpec((3, _EB), lambda i: (0, i))],
        out_specs=[pl.BlockSpec((_EB, NRB), lambda i: (i, 0)),
                   pl.BlockSpec((_EB, 16), lambda i: (i, 0))],
        out_shape=[jax.ShapeDtypeStruct((EP, NRB), jnp.float32),
                   jax.ShapeDtypeStruct((EP, 16), jnp.float32)],
    )(vecsT)


def _tc_edge_fwd(rb, sph, m0, ws0T, m1, ws1T):
    def body(rb_ref, sph_ref, a00, a01, a02, a03, w0T, b00, b01, b02, b03, w1T,
             p0_ref, p1_ref):
        rbv = rb_ref[...]
        sphv = sph_ref[...]
        for refs, wsT, out in (((a00, a01, a02, a03), w0T, p0_ref),
                               ((b00, b01, b02, b03), w1T, p1_ref)):
            _, _, _, rw = _mlp_fwd(rbv, refs[0][...], refs[1][...], refs[2][...], refs[3][...])
            se = _dot(sphv, wsT[...])
            out[...] = rw * se

    full = lambda i: (0, 0)
    specs = [pl.BlockSpec((_EB, NRB), lambda i: (i, 0)),
             pl.BlockSpec((_EB, 16), lambda i: (i, 0))]
    for shp in ((NRB, 64), (64, 64), (64, 64), (64, F), (16, 1),
                (NRB, 64), (64, 64), (64, 64), (64, F), (16, 1)):
        specs.append(pl.BlockSpec(shp, full))
    out_spec = pl.BlockSpec((_EB, F), lambda i: (i, 0))
    return pl.pallas_call(
        body, grid=(EP // _EB,), in_specs=specs,
        out_specs=[out_spec, out_spec],
        out_shape=[jax.ShapeDtypeStruct((EP, F), jnp.float32)] * 2,
    )(rb, sph, *m0, ws0T, *m1, ws1T)


def _tc_node_embed(species2d, w_emb, W_up0):
    def body(sp_ref, emb_ref, wu_ref, a0_ref):
        sp = sp_ref[...]
        oh = (sp == lax.broadcasted_iota(jnp.int32, (_NB, NS), 1)).astype(jnp.float32)
        nf0 = _dot(oh, emb_ref[...]) * jnp.float32(1.0 / math.sqrt(NS))
        a0_ref[...] = _dot(nf0, wu_ref[...])

    return pl.pallas_call(
        body, grid=(N2 // _NB,),
        in_specs=[pl.BlockSpec((_NB, 1), lambda i: (i, 0)),
                  pl.BlockSpec((NS, F), lambda i: (0, 0)),
                  pl.BlockSpec((F, F), lambda i: (0, 0))],
        out_specs=pl.BlockSpec((_NB, F), lambda i: (i, 0)),
        out_shape=jax.ShapeDtypeStruct((N2, F), jnp.float32),
    )(species2d, w_emb, W_up0)


def _onehot(sp):
    return (sp == lax.broadcasted_iota(jnp.int32, (sp.shape[0], NS), 1)).astype(jnp.float32)


def _species_mm(oh, x, wz_flat_ref):
    acc = jnp.zeros_like(x)
    for s in range(NS):
        acc = acc + oh[:, s:s + 1] * _dot(x, wz_flat_ref[s * F:(s + 1) * F, :])
    return acc


def _tc_node_l0(p00, p01, sp2d, W_down0, Wz0f, wsc0a, wsc0b, wsc0c, W_post0, W_up1, Wro0):
    def body(pa, pb, sp_ref, wd, wz, wa, wb, wc, wp, wu1, wro,
             nf1_ref, a1_ref, c0_ref, es0_ref):
        oh = _onehot(sp_ref[...])
        agg = (pa[...] + pb[...]) * jnp.float32(EPS)
        B0 = _dot(agg, wd[...])
        C0 = _species_mm(oh, B0, wz)
        wa_n = _dot(oh, wa[...])
        wb_n = _dot(oh, wb[...])
        wc_n = _dot(oh, wc[...])
        D0 = wa_n * C0 + wb_n * C0 * C0 + wc_n * C0 * C0 * C0
        nf1 = _dot(D0, wp[...])
        nf1_ref[...] = nf1
        a1_ref[...] = _dot(nf1, wu1[...])
        c0_ref[...] = C0
        es0_ref[...] = _dot(nf1, wro[...])

    nb = pl.BlockSpec((_NB, F), lambda i: (i, 0))
    full = lambda i: (0, 0)
    return pl.pallas_call(
        body, grid=(N2 // _NB,),
        in_specs=[nb, nb, pl.BlockSpec((_NB, 1), lambda i: (i, 0)),
                  pl.BlockSpec((F, F), full), pl.BlockSpec((NS * F, F), full),
                  pl.BlockSpec((NS, F), full), pl.BlockSpec((NS, F), full),
                  pl.BlockSpec((NS, F), full), pl.BlockSpec((F, F), full),
                  pl.BlockSpec((F, F), full), pl.BlockSpec((F, 1), full)],
        out_specs=[nb, nb, nb, pl.BlockSpec((_NB, 1), lambda i: (i, 0))],
        out_shape=[jax.ShapeDtypeStruct((N2, F), jnp.float32),
                   jax.ShapeDtypeStruct((N2, F), jnp.float32),
                   jax.ShapeDtypeStruct((N2, F), jnp.float32),
                   jax.ShapeDtypeStruct((N2, 1), jnp.float32)],
    )(p00, p01, sp2d, W_down0, Wz0f, wsc0a, wsc0b, wsc0c, W_post0, W_up1, Wro0)


def _tc_node_l1(p10, p11, nf1, sp2d, inde2d, es0, offsets2d, W_down1, Wz1f, Wz1Tf,
                wsc1a, wsc1b, wsc1c, W_post1, W_post1T, W_down1T, Wr1, Wr1T, Wr2, Wr2T):
    def body(pa, pb, nf_ref, sp_ref, ge_ref, es0_ref, off, wd, wz, wzT,
             wa, wb, wc, wp, wpT, wdT, wr1, wr1T, wr2, wr2T,
             gagg_ref, gnf_ref, eg_ref):
        oh = _onehot(sp_ref[...])
        nf = nf_ref[...]
        agg = (pa[...] + pb[...]) * jnp.float32(EPS)
        B1 = _dot(agg, wd[...])
        wa_n = _dot(oh, wa[...])
        wb_n = _dot(oh, wb[...])
        wc_n = _dot(oh, wc[...])
        D1 = wa_n * B1 + wb_n * B1 * B1 + wc_n * B1 * B1 * B1
        H1 = _dot(D1, wp[...]) + _species_mm(oh, nf, wz)
        u = _dot(H1, wr1[...])
        out1 = _dot(_silu(u), wr2[...])
        Ei = es0_ref[...] + out1 + _dot(oh, off[...])
        goh = (ge_ref[...] == lax.broadcasted_iota(jnp.int32, (_NB, G), 1)).astype(jnp.float32)
        eg_ref[...] = jnp.sum(goh * Ei, axis=0, keepdims=True)[None]
        gu = _dsilu(u) * wr2T[...]
        gH1 = _dot(gu, wr1T[...])
        gD1 = _dot(gH1, wpT[...])
        gnf_ref[...] = _species_mm(oh, gH1, wzT)
        gB1 = gD1 * (wa_n + 2.0 * wb_n * B1 + 3.0 * wc_n * B1 * B1)
        gagg_ref[...] = _dot(gB1, wdT[...]) * jnp.float32(EPS)

    nb = pl.BlockSpec((_NB, F), lambda i: (i, 0))
    n1 = pl.BlockSpec((_NB, 1), lambda i: (i, 0))
    full = lambda i: (0, 0)
    f3 = lambda i: (i, 0, 0)
    return pl.pallas_call(
        body, grid=(N2 // _NB,),
        in_specs=[nb, nb, nb, n1, n1, n1, pl.BlockSpec((NS, 1), full),
                  pl.BlockSpec((F, F), full), pl.BlockSpec((NS * F, F), full),
                  pl.BlockSpec((NS * F, F), full),
                  pl.BlockSpec((NS, F), full), pl.BlockSpec((NS, F), full),
                  pl.BlockSpec((NS, F), full), pl.BlockSpec((F, F), full),
                  pl.BlockSpec((F, F), full), pl.BlockSpec((F, F), full),
                  pl.BlockSpec((F, 16), full), pl.BlockSpec((16, F), full),
                  pl.BlockSpec((16, 1), full), pl.BlockSpec((1, 16), full)],
        out_specs=[nb, nb, pl.BlockSpec((1, 1, G), f3)],
        out_shape=[jax.ShapeDtypeStruct((N2, F), jnp.float32),
                   jax.ShapeDtypeStruct((N2, F), jnp.float32),
                   jax.ShapeDtypeStruct((N2 // _NB, 1, G), jnp.float32)],
    )(p10, p11, nf1, sp2d, inde2d, es0, offsets2d, W_down1, Wz1f, Wz1Tf,
      wsc1a, wsc1b, wsc1c, W_post1, W_post1T, W_down1T, Wr1, Wr1T, Wr2, Wr2T)


def _tc_node_l0_bwd(gnf1a, ga10, ga11, c0, sp2d, W_up1T, wro0row, W_post0T,
                    Wz0Tf, wsc0a, wsc0b, wsc0c, W_down0T):
    def body(gn_ref, ga, gb, c0_ref, sp_ref, wu1T, wro, wpT, wzT, wa, wb, wc, wdT,
             gagg_ref):
        oh = _onehot(sp_ref[...])
        C0 = c0_ref[...]
        gnf1 = gn_ref[...] + _dot(ga[...] + gb[...], wu1T[...]) + wro[...]
        gD0 = _dot(gnf1, wpT[...])
        wa_n = _dot(oh, wa[...])
        wb_n = _dot(oh, wb[...])
        wc_n = _dot(oh, wc[...])
        gC0 = gD0 * (wa_n + 2.0 * wb_n * C0 + 3.0 * wc_n * C0 * C0)
        gB0 = _species_mm(oh, gC0, wzT)
        gagg_ref[...] = _dot(gB0, wdT[...]) * jnp.float32(EPS)

    nb = pl.BlockSpec((_NB, F), lambda i: (i, 0))
    full = lambda i: (0, 0)
    return pl.pallas_call(
        body, grid=(N2 // _NB,),
        in_specs=[nb, nb, nb, nb, pl.BlockSpec((_NB, 1), lambda i: (i, 0)),
                  pl.BlockSpec((F, F), full), pl.BlockSpec((1, F), full),
                  pl.BlockSpec((F, F), full), pl.BlockSpec((NS * F, F), full),
                  pl.BlockSpec((NS, F), full), pl.BlockSpec((NS, F), full),
                  pl.BlockSpec((NS, F), full), pl.BlockSpec((F, F), full)],
        out_specs=nb,
        out_shape=jax.ShapeDtypeStruct((N2, F), jnp.float32),
    )(gnf1a, ga10, ga11, c0, sp2d, W_up1T, wro0row, W_post0T, Wz0Tf,
      wsc0a, wsc0b, wsc0c, W_down0T)


def _tc_edge_bwd(rb, sph, gp0, gp1, m0, m0T, ws0T, m1, m1T, ws1T):
    def body(rb_ref, sph_ref, g0_ref, g1_ref,
             a00, a01, a02, a03, ta0, ta1, ta2, ta3, w0T,
             b00, b01, b02, b03, tb0, tb1, tb2, tb3, w1T,
             grb_ref, gse_ref):
        rbv = rb_ref[...]
        sphv = sph_ref[...]
        grb = jnp.zeros((_EB, NRB), jnp.float32)
        gses = []
        for fw, bw, wsT, g_ref in (
                ((a00, a01, a02, a03), (ta0, ta1, ta2, ta3), w0T, g0_ref),
                ((b00, b01, b02, b03), (tb0, tb1, tb2, tb3), w1T, g1_ref)):
            u0, u1, u2, rw = _mlp_fwd(rbv, fw[0][...], fw[1][...], fw[2][...], fw[3][...])
            se = _dot(sphv, wsT[...])
            gp = g_ref[...]
            grw = gp * se
            gses.append(jnp.sum(gp * rw, axis=1, keepdims=True))
            ga2 = _dot(grw, bw[3][...])
            gu2 = ga2 * _dsilu(u2)
            ga1 = _dot(gu2, bw[2][...])
            gu1 = ga1 * _dsilu(u1)
            ga0 = _dot(gu1, bw[1][...])
            gu0 = ga0 * _dsilu(u0)
            grb = grb + _dot(gu0, bw[0][...])
        grb_ref[...] = grb
        z14 = jnp.zeros((_EB, 14), jnp.float32)
        gse_ref[...] = jnp.concatenate([gses[0], gses[1], z14], axis=1)

    eb = pl.BlockSpec((_EB, F), lambda i: (i, 0))
    full = lambda i: (0, 0)
    specs = [pl.BlockSpec((_EB, NRB), lambda i: (i, 0)),
             pl.BlockSpec((_EB, 16), lambda i: (i, 0)), eb, eb]
    for shp in ((NRB, 64), (64, 64), (64, 64), (64, F),
                (64, NRB), (64, 64), (64, 64), (F, 64), (16, 1)) * 2:
        specs.append(pl.BlockSpec(shp, full))
    return pl.pallas_call(
        body, grid=(EP // _EB,), in_specs=specs,
        out_specs=[pl.BlockSpec((_EB, NRB), lambda i: (i, 0)),
                   pl.BlockSpec((_EB, 16), lambda i: (i, 0))],
        out_shape=[jax.ShapeDtypeStruct((EP, NRB), jnp.float32),
                   jax.ShapeDtypeStruct((EP, 16), jnp.float32)],
    )(rb, sph, gp0, gp1, *m0, *m0T, ws0T, *m1, *m1T, ws1T)


def _tc_force(vecsT, grb, gse2, maskT, ws0col, ws1col):
    def body(v_ref, grb_ref, gse_ref, mk_ref, w0c, w1c, gvp_ref, gvn_ref):
        vT = v_ref[...]
        r, _, drbT, _, vh = _geom_T(vT, True)
        grbT = jnp.swapaxes(grb_ref[...], 0, 1)
        gseT = jnp.swapaxes(gse_ref[...], 0, 1)
        gr = jnp.sum(grbT * drbT, axis=0, keepdims=True)
        gsphT = w0c[...] * gseT[0:1, :] + w1c[...] * gseT[1:2, :]
        xh, yh, zh = vh
        gx = gsphT[1:2] + gsphT[4:5] * yh + gsphT[7:8] * zh + 2.0 * xh * gsphT[8:9]
        gy = gsphT[2:3] + gsphT[4:5] * xh + gsphT[5:6] * zh - 2.0 * yh * gsphT[8:9]
        gz = gsphT[3:4] + gsphT[5:6] * yh + 6.0 * zh * gsphT[6:7] + gsphT[7:8] * xh
        rp = r + 1e-9
        vx = vT[0:1, :]
        vy = vT[1:2, :]
        vz = vT[2:3, :]
        vdg = (vx * gx + vy * gy + vz * gz) / (rp * rp * r)
        s1 = gr / r
        mk = mk_ref[...]
        gvx = (s1 * vx + gx / rp - vx * vdg) * mk
        gvy = (s1 * vy + gy / rp - vy * vdg) * mk
        gvz = (s1 * vz + gz / rp - vz * vdg) * mk
        z13 = jnp.zeros((13, _EB), jnp.float32)
        gvT = jnp.concatenate([gvx, gvy, gvz, z13], axis=0)
        gvp_ref[...] = jnp.swapaxes(gvT, 0, 1)
        gvn_ref[...] = jnp.swapaxes(-gvT, 0, 1)

    o16 = pl.BlockSpec((_EB, 16), lambda i: (i, 0))
    full = lambda i: (0, 0)
    return pl.pallas_call(
        body, grid=(EP // _EB,),
        in_specs=[pl.BlockSpec((3, _EB), lambda i: (0, i)),
                  pl.BlockSpec((_EB, NRB), lambda i: (i, 0)), o16,
                  pl.BlockSpec((1, _EB), lambda i: (0, i)),
                  pl.BlockSpec((16, 1), full), pl.BlockSpec((16, 1), full)],
        out_specs=[o16, o16],
        out_shape=[jax.ShapeDtypeStruct((EP, 16), jnp.float32)] * 2,
    )(vecsT, grb, gse2, maskT, ws0col, ws1col)


_MESH = dict(core_axis_name="c", subcore_axis_name="s")


def _wid():
    return lax.axis_index("s") * 2 + lax.axis_index("c")


def _split_sched(nch0=_NCH0):
    cid = lax.axis_index("c")
    sid = lax.axis_index("s")
    nchp = jnp.where(cid == 0, nch0 // _PH, (_NCH0 + _NCH1 - nch0) // _PH)
    base = sid * (_NCH0 + _NCH1) + cid * nch0

    def tab_base(ph):
        return base + ph * nchp

    return nchp, tab_base


def _mul_rows(dst, src, nrows):
    @plsc.parallel_loop(0, nrows, unroll=2)
    def mul(e):
        for j in range(F // 16):
            sl = pl.ds(j * 16, 16)
            dst[e, sl] = dst[e, sl] * src[e, sl]


def _sc_fwd(A, P, ia2, ib2, zeros_nf):
    mesh = plsc.VectorSubcoreMesh(**_MESH)

    @functools.partial(
        pl.kernel, mesh=mesh,
        out_type=jax.ShapeDtypeStruct((2, N2, F), jnp.float32),
        scratch_types=[
            pltpu.VMEM((_NCHP0, _B), jnp.int32),
            pltpu.VMEM((_NCHP0, _B), jnp.int32),
            pltpu.VMEM((_B, F), jnp.float32),
            pltpu.VMEM((_B, F), jnp.float32),
            pltpu.VMEM_SHARED((N2, F), jnp.float32),
            pltpu.SemaphoreType.DMA,
            pltpu.SemaphoreType.DMA,
        ],
    )
    def k(a_hbm, p_hbm, ia_hbm, ib_hbm, z_hbm, out_hbm, ia_v, ib_v, rows_v, p_v,
          acc, sem, semp):
        cid = lax.axis_index("c")
        sid = lax.axis_index("s")
        rbase = sid * _RPS
        pltpu.sync_copy(z_hbm.at[pl.ds(rbase, _RPS)], acc.at[pl.ds(rbase, _RPS)])
        plsc.subcore_barrier()
        nchp, tab_base = _split_sched()

        for ph in range(_PH):
            tb = tab_base(ph)
            pltpu.sync_copy(ia_hbm.at[pl.ds(tb, _NCHP0)], ia_v)
            pltpu.sync_copy(ib_hbm.at[pl.ds(tb, _NCHP0)], ib_v)

            def chunk(ci, _):
                b = (tb + ci) * _B
                cp = pltpu.async_copy(p_hbm.at[pl.ds(b, _B)], p_v, semp)
                pltpu.async_copy(a_hbm.at[ia_v.at[ci]], rows_v, sem).wait()
                cp.wait()
                _mul_rows(p_v, rows_v, _B)
                pltpu.sync_copy(p_v, acc.at[ib_v.at[ci]], add=True)
                return 0

            lax.fori_loop(0, nchp, chunk, 0)
        plsc.subcore_barrier()
        pltpu.sync_copy(acc.at[pl.ds(rbase, _RPS)],
                        out_hbm.at[cid, pl.ds(rbase, _RPS)])

    return k(A, P, ia2, ib2, zeros_nf)


def _sc_bwd1(Gs, A, P, ia2, ib2, zeros_nf):
    mesh = plsc.VectorSubcoreMesh(**_MESH)

    @functools.partial(
        pl.kernel, mesh=mesh,
        out_type=[jax.ShapeDtypeStruct((EP, F), jnp.float32),
                  jax.ShapeDtypeStruct((2, N2, F), jnp.float32)],
        scratch_types=[
            pltpu.VMEM((_NCHP0, _B), jnp.int32),
            pltpu.VMEM((_NCHP0, _B), jnp.int32),
            pltpu.VMEM((_B, F), jnp.float32),
            pltpu.VMEM((_B, F), jnp.float32),
            pltpu.VMEM_SHARED((N2, F), jnp.float32),
            pltpu.SemaphoreType.DMA,
            pltpu.SemaphoreType.DMA,
            pltpu.SemaphoreType.DMA,
        ],
    )
    def k(g_hbm, a_hbm, p_hbm, ia_hbm, ib_hbm, z_hbm, gp_hbm, out_hbm,
          ia_v, ib_v, q_v, a_v, acc, sem, semq, semp):
        cid = lax.axis_index("c")
        sid = lax.axis_index("s")
        rbase = sid * _RPS
        pltpu.sync_copy(z_hbm.at[pl.ds(rbase, _RPS)], acc.at[pl.ds(rbase, _RPS)])
        plsc.subcore_barrier()
        nchp, tab_base = _split_sched()

        for ph in range(_PH):
            tb = tab_base(ph)
            pltpu.sync_copy(ia_hbm.at[pl.ds(tb, _NCHP0)], ia_v)
            pltpu.sync_copy(ib_hbm.at[pl.ds(tb, _NCHP0)], ib_v)

            def chunk(ci, _):
                b = (tb + ci) * _B
                cq = pltpu.async_copy(g_hbm.at[ib_v.at[ci]], q_v, semq)
                pltpu.async_copy(a_hbm.at[ia_v.at[ci]], a_v, sem).wait()
                cq.wait()
                _mul_rows(a_v, q_v, _B)
                pltpu.sync_copy(a_v, gp_hbm.at[pl.ds(b, _B)])
                pltpu.async_copy(p_hbm.at[pl.ds(b, _B)], a_v, semp).wait()
                _mul_rows(a_v, q_v, _B)
                pltpu.sync_copy(a_v, acc.at[ia_v.at[ci]], add=True)
                return 0

            lax.fori_loop(0, nchp, chunk, 0)
        plsc.subcore_barrier()
        pltpu.sync_copy(acc.at[pl.ds(rbase, _RPS)],
                        out_hbm.at[cid, pl.ds(rbase, _RPS)])

    return k(Gs, A, P, ia2, ib2, zeros_nf)


def _sc_bwd0(Gs, A, ia2, ib2):
    mesh = plsc.VectorSubcoreMesh(**_MESH)

    @functools.partial(
        pl.kernel, mesh=mesh,
        out_type=jax.ShapeDtypeStruct((EP, F), jnp.float32),
        scratch_types=[
            pltpu.VMEM((_NCHP0, _B), jnp.int32),
            pltpu.VMEM((_NCHP0, _B), jnp.int32),
            pltpu.VMEM((_B, F), jnp.float32),
            pltpu.VMEM((_B, F), jnp.float32),
            pltpu.SemaphoreType.DMA,
            pltpu.SemaphoreType.DMA,
        ],
    )
    def k(g_hbm, a_hbm, ia_hbm, ib_hbm, gp_hbm, ia_v, ib_v, q_v, a_v, sem, semq):
        nchp, tab_base = _split_sched()

        for ph in range(_PH):
            tb = tab_base(ph)
            pltpu.sync_copy(ia_hbm.at[pl.ds(tb, _NCHP0)], ia_v)
            pltpu.sync_copy(ib_hbm.at[pl.ds(tb, _NCHP0)], ib_v)

            def chunk(ci, _):
                b = (tb + ci) * _B
                cq = pltpu.async_copy(g_hbm.at[ib_v.at[ci]], q_v, semq)
                pltpu.async_copy(a_hbm.at[ia_v.at[ci]], a_v, sem).wait()
                cq.wait()
                _mul_rows(a_v, q_v, _B)
                pltpu.sync_copy(a_v, gp_hbm.at[pl.ds(b, _B)])
                return 0

            lax.fori_loop(0, nchp, chunk, 0)

    return k(Gs, A, ia2, ib2)


def _sc_forces(idx2, val2, zeros_nf):
    mesh = plsc.VectorSubcoreMesh(**_MESH)

    @functools.partial(
        pl.kernel, mesh=mesh,
        out_type=jax.ShapeDtypeStruct((2, N2, F), jnp.float32),
        scratch_types=[
            pltpu.VMEM((_NCH2 // _PH, _B2), jnp.int32),
            pltpu.VMEM((_B2, 16), jnp.float32),
            pltpu.VMEM((_B2, F), jnp.float32),
            pltpu.VMEM_SHARED((N2, F), jnp.float32),
            pltpu.SemaphoreType.DMA,
        ],
    )
    def k(ix_hbm, v_hbm, z_hbm, out_hbm, ix_v, v_v, w_v, acc, semv):
        cid = lax.axis_index("c")
        sid = lax.axis_index("s")
        rbase = sid * _RPS
        pltpu.sync_copy(z_hbm.at[pl.ds(rbase, _RPS)], acc.at[pl.ds(rbase, _RPS)])
        w = _wid()
        zv = jnp.zeros((16,), jnp.float32)

        @plsc.parallel_loop(0, _B2, unroll=2)
        def zrow(e):
            for j in range(F // 16):
                w_v[e, pl.ds(j * 16, 16)] = zv
        plsc.subcore_barrier()
        nchp2 = _NCH2 // _PH

        for ph in range(_PH):
            pltpu.sync_copy(ix_hbm.at[pl.ds(w * _NCH2 + ph * nchp2, nchp2)], ix_v)
            ebase = w * _EPW2 + ph * nchp2 * _B2

            def chunk(ci, _):
                b = ebase + ci * _B2
                pltpu.async_copy(v_hbm.at[pl.ds(b, _B2)], v_v, semv).wait()

                @plsc.parallel_loop(0, _B2, unroll=4)
                def crow(e):
                    w_v[e, pl.ds(0, 16)] = v_v[e, pl.ds(0, 16)]
                pltpu.sync_copy(w_v, acc.at[ix_v.at[ci]], add=True)
                return 0

            lax.fori_loop(0, nchp2, chunk, 0)
        plsc.subcore_barrier()
        pltpu.sync_copy(acc.at[pl.ds(rbase, _RPS)],
                        out_hbm.at[cid, pl.ds(rbase, _RPS)])

    return k(idx2, val2, zeros_nf)


def kernel(nn_vecs, species, inda, indb, inde, nats, mask, w_emb, offsets,
           l0_W_up, l0_mlp0, l0_mlp1, l0_mlp2, l0_mlp3, l0_w_s, l0_W_down,
           l0_Wz, l0_w_sc, l0_W_post,
           l1_W_up, l1_mlp0, l1_mlp1, l1_mlp2, l1_mlp3, l1_w_s, l1_W_down,
           l1_Wz, l1_w_sc, l1_W_post,
           Wro0, Wr1, Wr2):
    f32 = jnp.float32
    sp2d = jnp.pad(species.astype(jnp.int32), (0, N2 - N)).reshape(N2, 1)
    ia = jnp.pad(inda.astype(jnp.int32), (0, EP - E), constant_values=N)
    ib = jnp.pad(indb.astype(jnp.int32), (0, EP - E), constant_values=N)
    ia2 = jnp.pad(ia.reshape(EP // _B, _B), ((0, _CTAB - EP // _B), (0, 0)))
    ib2 = jnp.pad(ib.reshape(EP // _B, _B), ((0, _CTAB - EP // _B), (0, 0)))
    inde2d = jnp.pad(inde.astype(jnp.int32), (0, N2 - N), constant_values=G).reshape(N2, 1)
    maskT = jnp.pad(mask.astype(f32), (0, EP - E)).reshape(1, EP)
    vecsT = jnp.pad(nn_vecs, ((0, EP - E), (0, 0)), constant_values=1.0).T
    zeros_nf = jnp.zeros((N2, F), f32)

    def pad16(w):
        return jnp.concatenate([w, jnp.zeros((7,), f32)]).reshape(16, 1)

    ws0T, ws1T = pad16(l0_w_s), pad16(l1_w_s)
    m0 = (l0_mlp0, l0_mlp1, l0_mlp2, l0_mlp3)
    m1 = (l1_mlp0, l1_mlp1, l1_mlp2, l1_mlp3)
    m0T = tuple(w.T for w in m0)
    m1T = tuple(w.T for w in m1)
    Wz0f = l0_Wz.reshape(NS * F, F)
    Wz1f = l1_Wz.reshape(NS * F, F)
    Wz0Tf = jnp.swapaxes(l0_Wz, 1, 2).reshape(NS * F, F)
    Wz1Tf = jnp.swapaxes(l1_Wz, 1, 2).reshape(NS * F, F)
    wsc0a, wsc0b, wsc0c = l0_w_sc[:, 0], l0_w_sc[:, 1], l0_w_sc[:, 2]
    wsc1a, wsc1b, wsc1c = l1_w_sc[:, 0], l1_w_sc[:, 1], l1_w_sc[:, 2]

    rb, sph = _tc_geom(vecsT)
    P0, P1 = _tc_edge_fwd(rb, sph, m0, ws0T, m1, ws1T)
    A0 = _tc_node_embed(sp2d, w_emb, l0_W_up)
    agg0p = _sc_fwd(A0, P0, ia2, ib2, zeros_nf)
    nf1, A1, C0, Es0 = _tc_node_l0(agg0p[0], agg0p[1], sp2d, l0_W_down, Wz0f,
                                   wsc0a, wsc0b, wsc0c, l0_W_post, l1_W_up, Wro0)
    agg1p = _sc_fwd(A1, P1, ia2, ib2, zeros_nf)
    gagg1, gnf1a, egp = _tc_node_l1(
        agg1p[0], agg1p[1], nf1, sp2d, inde2d, Es0, offsets.reshape(NS, 1),
        l1_W_down, Wz1f, Wz1Tf, wsc1a, wsc1b, wsc1c, l1_W_post, l1_W_post.T,
        l1_W_down.T, Wr1, Wr1.T, Wr2, Wr2.T)
    Eg = jnp.sum(egp, axis=(0, 1))

    gP1, ga1p = _sc_bwd1(gagg1, A1, P1, ia2, ib2, zeros_nf)
    gagg0 = _tc_node_l0_bwd(gnf1a, ga1p[0], ga1p[1], C0, sp2d, l1_W_up.T,
                            Wro0.reshape(1, F), l0_W_post.T, Wz0Tf,
                            wsc0a, wsc0b, wsc0c, l0_W_down.T)
    gP0 = _sc_bwd0(gagg0, A0, ia2, ib2)
    grb, gse2 = _tc_edge_bwd(rb, sph, gP0, gP1, m0, m0T, ws0T, m1, m1T, ws1T)
    gvp, gvn = _tc_force(vecsT, grb, gse2, maskT, ws0T, ws1T)
    idx2 = jnp.concatenate([ia, ib]).reshape((2 * EP) // _B2, _B2)
    val2 = jnp.concatenate([gvp, gvn], axis=0)
    fp = _sc_forces(idx2, val2, zeros_nf)
    Fn = (fp[0] + fp[1])[:N, :3]
    return (Eg, Fn)
